# Initial kernel scaffold; baseline (speedup 1.0000x reference)
import jax
import jax.numpy as jnp
from jax.experimental import pallas as pl


def kernel(x):
    raise NotImplementedError("write your pallas kernel here")


# speedup vs baseline: 11.1808x; 11.1808x over previous
import functools

import jax
import jax.numpy as jnp
from jax import lax
from jax.experimental import pallas as pl
from jax.experimental.pallas import tpu as pltpu
from jax.experimental.pallas import tpu_sc as plsc

R = 128
N = 32768
K = (N - 1) // 2
L = 16
NV = N // L
NW = 32
RPW = R // NW
U = 8
HI0 = 0x3F7FFFFF

_mesh = plsc.VectorSubcoreMesh(core_axis_name="c", subcore_axis_name="s")


@functools.partial(
    pl.kernel,
    mesh=_mesh,
    out_type=jax.ShapeDtypeStruct((NW, L), jnp.float32),
    compiler_params=pltpu.CompilerParams(needs_layout_passes=False),
    scratch_types=[
        pltpu.VMEM((N,), jnp.float32),
        pltpu.VMEM((L,), jnp.float32),
    ],
)
def _soft_majority_sc(x_hbm, out_hbm, row_v, res_v):
    wid = lax.axis_index("s") * 2 + lax.axis_index("c")
    lane = lax.iota(jnp.int32, 16)
    zero_i = jnp.zeros((L,), jnp.int32)
    k_vec = jnp.full((L,), K, jnp.int32)
    res = jnp.zeros((L,), jnp.float32)

    for j in range(RPW):
        row = wid * RPW + j
        pltpu.sync_copy(x_hbm.at[row], row_v)

        def sum_body(i, acc):
            b = i * (L * U)
            for u in range(U):
                acc = acc + row_v[pl.ds(b + u * L, L)]
            return acc

        acc = lax.fori_loop(0, NV // U, sum_body, jnp.zeros((L,), jnp.float32))
        total = jnp.float32(0.0)
        for t in range(L):
            total = total + acc[t]
        mean = total * (1.0 / N)

        def pass_body(_, lohi):
            lo, hi = lohi
            mid = (lo + hi) >> 1
            mid_f = lax.bitcast_convert_type(mid, jnp.float32)

            def cnt_body(i, cacc):
                b = i * (L * U)
                for u in range(U):
                    v = row_v[pl.ds(b + u * L, L)]
                    cacc = cacc + plsc.all_reduce_population_count(v <= mid_f)
                return cacc

            c = lax.fori_loop(0, NV // U, cnt_body, zero_i)
            go_low = c > k_vec
            lo = jnp.where(go_low, lo, mid + 1)
            hi = jnp.where(go_low, mid, hi)
            return (lo, hi)

        lo, _ = lax.fori_loop(0, 30, pass_body,
                              (zero_i, jnp.full((L,), HI0, jnp.int32)))

        m_bit = lax.bitcast_convert_type(lo, jnp.float32)
        margin = jnp.abs(m_bit - 0.5)
        md = mean * margin
        rep = jnp.where(m_bit > 0.5, 0.5 + md, m_bit + md)
        res = jnp.where(lane == j, rep, res)

    res_v[...] = res
    pltpu.sync_copy(res_v, out_hbm.at[wid])


def kernel(x):
    padded = _soft_majority_sc(x)
    return padded[:, :RPW].reshape(R)
